# Initial kernel scaffold; baseline (speedup 1.0000x reference)
import jax
import jax.numpy as jnp
from jax.experimental import pallas as pl


def kernel(x, k, v, s, abs_x, deg, idx, Wq, Wk, Wv):
    raise NotImplementedError("write your pallas kernel here")


# speedup vs baseline: 219.8469x; 219.8469x over previous
import functools

import jax
import jax.numpy as jnp
from jax.experimental import pallas as pl

B, CIN, C, N, K, L = 8, 256, 128, 256, 32, 2
KL = K * L
TOPK = 5 * L
CN = 64
NC = N // CN
R = CN * K
RM = CN * KL

_NEG = -3e38
_BIG = 1 << 30


def _body(x_ref, k_ref, v_ref, st_ref, ax_ref, keep_ref, idxq2_ref,
          cidx2_ref, wq_ref, wk_ref, wv_ref,
          out_ref, kq_ref, vq_ref, sc_ref):
    nc = pl.program_id(1)
    f32 = jnp.float32

    xb = x_ref[0]
    kq_cm = jnp.dot(wk_ref[...], xb, preferred_element_type=f32)
    vq_cm = jnp.dot(wv_ref[...], xb, preferred_element_type=f32)
    q_t = jax.lax.dot_general(ax_ref[0], wq_ref[...],
                              (((1,), (1,)), ((), ())),
                              preferred_element_type=f32)
    kq_ref[0] = kq_cm.reshape(C, CN, K)
    vq_ref[0] = vq_cm.reshape(C, CN, K)
    kq3 = kq_cm.T.reshape(CN, K, C)
    vq3 = vq_cm.T.reshape(CN, K, C)

    lg = q_t[:, None, :] * kq3
    mx = jnp.max(lg, axis=1, keepdims=True)
    e = jnp.exp(lg - mx)
    outq = e / jnp.sum(e, axis=1, keepdims=True)
    oq = jnp.sum(outq * vq3, axis=1)

    keep = keep_ref[0]
    w_sc = (outq * keep[:, :, None]).reshape(R, C)
    ohs = (idxq2_ref[...] ==
           jax.lax.broadcasted_iota(jnp.int32, (R, N), 1)).astype(f32)
    sc_cm = jnp.dot(w_sc.T, ohs, preferred_element_type=f32,
                    precision=jax.lax.Precision.HIGHEST)

    @pl.when(nc == 0)
    def _():
        sc_ref[0] = jnp.zeros_like(sc_ref[0])
    sc_ref[0] += sc_cm

    ohm = (cidx2_ref[...] ==
           jax.lax.broadcasted_iota(jnp.int32, (RM, N * L), 1)).astype(f32)
    s_mem = jnp.dot(ohm, st_ref[0], preferred_element_type=f32,
                    precision=jax.lax.Precision.HIGHEST).reshape(CN, KL, C)

    jio = jax.lax.broadcasted_iota(jnp.int32, (CN, KL, C), 1)
    taken = jnp.zeros((CN, KL, C), dtype=jnp.bool_)
    for _ in range(TOPK):
        cur = jnp.where(taken, _NEG, s_mem)
        m = jnp.max(cur, axis=1, keepdims=True)
        cand = jnp.where(cur == m, jio, _BIG)
        jmin = jnp.min(cand, axis=1, keepdims=True)
        taken = taken | (jio == jmin)

    kmem = k_ref[0].T.reshape(CN, KL, C)
    vmem = v_ref[0].T.reshape(CN, KL, C)
    lgm = q_t[:, None, :] * kmem
    mx2 = jnp.max(jnp.where(taken, lgm, _NEG), axis=1, keepdims=True)
    e2 = jnp.where(taken, jnp.exp(lgm - mx2), f32(0))
    om = jnp.sum(e2 * vmem, axis=1) / jnp.sum(e2, axis=1)

    out_ref[0] = jnp.concatenate([oq, om], axis=1)


@jax.jit
def kernel(x, k, v, s, abs_x, deg, idx, Wq, Wk, Wv):
    del deg
    f32 = jnp.float32
    x3 = x.reshape(B, CIN, N * K)
    k3 = k.reshape(B, C, N * KL)
    v3 = v.reshape(B, C, N * KL)
    st2 = jnp.transpose(s, (0, 2, 3, 1)).reshape(B, N * L, C)
    ax3 = jnp.transpose(abs_x.reshape(B, CIN // 2, N), (0, 2, 1))
    idxq3 = idx[:, :, :, L]
    idxq2 = idxq3.reshape(B * N * K, 1)

    r = jnp.arange(B * N * K, dtype=jnp.int32)
    keys = (r // K) * N + idxq2[:, 0]
    sk, sv = jax.lax.sort((keys, r.astype(f32)), dimension=0, num_keys=1,
                          is_stable=False)
    is_last = jnp.concatenate(
        [sk[1:] != sk[:-1], jnp.ones((1,), jnp.bool_)]).astype(f32)
    keep3 = jnp.zeros((B * N * K,), f32).at[sv.astype(jnp.int32)].set(
        is_last, unique_indices=True, mode="promise_in_bounds"
    ).reshape(B, N, K)
    cidx2 = (idx[:, :, :, :L] * L +
             jnp.arange(L, dtype=jnp.int32)).reshape(B * N * K * L, 1)

    grid = (B, NC)
    out3, kq4, vq4, sc3 = pl.pallas_call(
        _body,
        grid=grid,
        in_specs=[
            pl.BlockSpec((1, CIN, R), lambda b, nc: (b, 0, nc)),
            pl.BlockSpec((1, C, RM), lambda b, nc: (b, 0, nc)),
            pl.BlockSpec((1, C, RM), lambda b, nc: (b, 0, nc)),
            pl.BlockSpec((1, N * L, C), lambda b, nc: (b, 0, 0)),
            pl.BlockSpec((1, CN, CIN // 2), lambda b, nc: (b, nc, 0)),
            pl.BlockSpec((1, CN, K), lambda b, nc: (b, nc, 0)),
            pl.BlockSpec((R, 1), lambda b, nc: (b * NC + nc, 0)),
            pl.BlockSpec((RM, 1), lambda b, nc: (b * NC + nc, 0)),
            pl.BlockSpec((C, CIN // 2), lambda b, nc: (0, 0)),
            pl.BlockSpec((C, CIN), lambda b, nc: (0, 0)),
            pl.BlockSpec((C, CIN), lambda b, nc: (0, 0)),
        ],
        out_specs=[
            pl.BlockSpec((1, CN, 2 * C), lambda b, nc: (b, nc, 0)),
            pl.BlockSpec((1, C, CN, K), lambda b, nc: (b, 0, nc, 0)),
            pl.BlockSpec((1, C, CN, K), lambda b, nc: (b, 0, nc, 0)),
            pl.BlockSpec((1, C, N), lambda b, nc: (b, 0, 0)),
        ],
        out_shape=[
            jax.ShapeDtypeStruct((B, N, 2 * C), f32),
            jax.ShapeDtypeStruct((B, C, N, K), f32),
            jax.ShapeDtypeStruct((B, C, N, K), f32),
            jax.ShapeDtypeStruct((B, C, N), f32),
        ],
    )(x3, k3, v3, st2, ax3, keep3, idxq2, cidx2, Wq, Wk, Wv)

    return (jnp.transpose(out3, (0, 2, 1)).reshape(B, 2 * C, N, 1),
            kq4.reshape(B, C, N, K, 1),
            vq4.reshape(B, C, N, K, 1),
            sc3.reshape(B, C, N, 1))
